# baseline (speedup 1.0000x reference)
import functools

import jax
import jax.numpy as jnp
from jax import lax
from jax.experimental import pallas as pl
from jax.experimental.pallas import tpu as pltpu
from jax.experimental.pallas import tpu_sc as plsc

_L = 16
_T = 64
_NB = 4
_SENT = -1


def _bf16_round(v):
    b = jax.lax.bitcast_convert_type(v, jnp.uint32)
    b = (b + jnp.uint32(0x7FFF) + ((b >> jnp.uint32(16)) & jnp.uint32(1)))
    b = b & jnp.uint32(0xFFFF0000)
    return jax.lax.bitcast_convert_type(b, jnp.float32)


def _sampler_body(H, W, C, rows_per_w, n_tiles,
                  theta_hbm, img_hbm, out_hbm,
                  theta_v, out_v, *bufs):
    HW = H * W
    per_set = 12
    sets = []
    for s in range(_NB):
        grp = bufs[s * per_set:(s + 1) * per_set]
        sets.append((grp[0:4], grp[4:8], grp[8:12], bufs[_NB * per_set + s]))

    wid = lax.axis_index("s") * 2 + lax.axis_index("c")
    row0 = wid * rows_per_w
    b = row0 // HW
    base = b * HW
    p0 = row0 - base

    pltpu.sync_copy(theta_hbm, theta_v)
    tbase = jnp.full((16,), b * 6, jnp.int32)

    def _tsplat(k):
        return _bf16_round(plsc.load_gather(theta_v, [tbase + k]))

    t00 = _tsplat(0)
    t01 = _tsplat(1)
    t02 = _tsplat(2)
    t10 = _tsplat(3)
    t11 = _tsplat(4)
    t12 = _tsplat(5)

    def compute_iw(g, s):
        idx, wts, _, _ = sets[s]
        ia_i, ib_i, ic_i, id_i = idx
        wa_v, wb_v, wc_v, wd_v = wts
        pstart = p0 + g * _T
        for u in range(_T // _L):
            p = (pstart + u * _L) + lax.iota(jnp.int32, 16)
            i = p // W
            j = p - i * W
            xn = _bf16_round(j.astype(jnp.float32) * jnp.float32(2.0 / (W - 1)) - 1.0)
            yn = _bf16_round(i.astype(jnp.float32) * jnp.float32(2.0 / (H - 1)) - 1.0)
            xs = t00 * xn + t01 * yn + t02
            ys = t10 * xn + t11 * yn + t12
            x = 0.5 * (xs + 1.0) * jnp.float32(W - 1)
            y = 0.5 * (ys + 1.0) * jnp.float32(H - 1)
            xt = x.astype(jnp.int32)
            yt = y.astype(jnp.int32)
            x0i = jnp.where(x < xt.astype(jnp.float32), xt - 1, xt)
            y0i = jnp.where(y < yt.astype(jnp.float32), yt - 1, yt)
            x0f = x0i.astype(jnp.float32)
            y0f = y0i.astype(jnp.float32)
            x0c = jnp.clip(x0i, 0, W - 1)
            x1c = jnp.clip(x0i + 1, 0, W - 1)
            y0c = jnp.clip(y0i, 0, H - 1)
            y1c = jnp.clip(y0i + 1, 0, H - 1)
            wx1 = (x0f + 1.0) - x
            wx0 = x - x0f
            wy1 = (y0f + 1.0) - y
            wy0 = y - y0f
            wa = wx1 * wy1
            wb = wx1 * wy0
            wc = wx0 * wy1
            wd = wx0 * wy0
            xcl = (x0i < 0) | (x0i >= W - 1)
            ycl = (y0i < 0) | (y0i >= H - 1)
            zero = jnp.zeros((16,), jnp.float32)
            wb_x = wb + jnp.where(xcl, wd, zero)
            wa_f = (wa + jnp.where(xcl, wc, zero)
                    + jnp.where(ycl, wb_x, zero))
            wb_f = jnp.where(ycl, zero, wb_x)
            wc_f = jnp.where(xcl, zero, wc + jnp.where(ycl, wd, zero))
            wd_f = jnp.where(xcl | ycl, zero, wd)
            sent = jnp.full((16,), _SENT, jnp.int32)
            sl = pl.ds(u * _L, _L)
            ia_i[sl] = base + y0c * W + x0c
            ib_i[sl] = jnp.where(ycl, sent, base + y1c * W + x0c)
            ic_i[sl] = jnp.where(xcl, sent, base + y0c * W + x1c)
            id_i[sl] = jnp.where(xcl | ycl, sent, base + y1c * W + x1c)
            wa_v[sl] = wa_f
            wb_v[sl] = wb_f
            wc_v[sl] = wc_f
            wd_v[sl] = wd_f

    def fire(s):
        idx, _, gat, sm = sets[s]
        for k in range(4):
            src = img_hbm.at[plsc.Indices(idx[k], ignored_value=_SENT)]
            pltpu.async_copy(src, gat[k], sm)

    def drain(s):
        idx, _, gat, sm = sets[s]
        for k in range(4):
            src = img_hbm.at[plsc.Indices(idx[k], ignored_value=_SENT)]
            pltpu.make_async_copy(src, gat[k], sm).wait()

    def combine_and_store(g, s):
        _, wts, gat, _ = sets[s]
        wa_v, wb_v, wc_v, wd_v = wts
        ga_v, gb_v, gc_v, gd_v = gat

        def row_body(t, c2):
            tt = jnp.full((16,), t, jnp.int32)
            wa = plsc.load_gather(wa_v, [tt])
            wb = plsc.load_gather(wb_v, [tt])
            wc = plsc.load_gather(wc_v, [tt])
            wd = plsc.load_gather(wd_v, [tt])
            for c0 in range(C // _L):
                cs = pl.ds(c0 * _L, _L)
                out_v[t, cs] = (wa * ga_v[t, cs] + wb * gb_v[t, cs]
                                + wc * gc_v[t, cs] + wd * gd_v[t, cs])
            return c2

        lax.fori_loop(0, _T, row_body, 0, unroll=2)
        pltpu.sync_copy(out_v, out_hbm.at[pl.ds(row0 + g * _T, _T)])

    def zero_body(t, c2):
        for s in range(_NB):
            for k in range(4):
                for c0 in range(C // _L):
                    sets[s][2][k][t, pl.ds(c0 * _L, _L)] = (
                        jnp.zeros((16,), jnp.float32))
        return c2

    lax.fori_loop(0, _T, zero_body, 0)

    for k in range(_NB - 1):
        compute_iw(k, k)
        fire(k)

    def ring_body(gg, carry):
        g = gg * _NB
        for p in range(_NB):
            tile = g + p
            s = p
            drain(s)
            nt = tile + _NB - 1
            s2 = (p + _NB - 1) % _NB

            @pl.when(nt < n_tiles)
            def _():
                compute_iw(nt, s2)
                fire(s2)

            combine_and_store(tile, s)
        return carry

    lax.fori_loop(0, n_tiles // _NB, ring_body, 0)


def kernel(theta, image):
    B, H, W, C = image.shape
    info = plsc.get_sparse_core_info()
    nw = info.num_cores * info.num_subcores
    total = B * H * W
    assert total % (nw * _T) == 0 and C % _L == 0
    rows_per_w = total // nw
    n_tiles = rows_per_w // _T
    assert n_tiles % _NB == 0

    img_flat = image.reshape(total, C)
    mesh = plsc.VectorSubcoreMesh(core_axis_name="c", subcore_axis_name="s")

    def bufset():
        return ([pltpu.VMEM((_T,), jnp.int32) for _ in range(4)]
                + [pltpu.VMEM((_T,), jnp.float32) for _ in range(4)]
                + [pltpu.VMEM((_T, C), jnp.float32) for _ in range(4)])

    scratch = [pltpu.VMEM((B * 6,), jnp.float32),
               pltpu.VMEM((_T, C), jnp.float32)]
    for _ in range(_NB):
        scratch += bufset()
    scratch += [pltpu.SemaphoreType.DMA] * _NB

    run = pl.kernel(
        functools.partial(_sampler_body, H, W, C, rows_per_w, n_tiles),
        out_type=jax.ShapeDtypeStruct((total, C), jnp.float32),
        mesh=mesh,
        scratch_types=scratch,
        compiler_params=pltpu.CompilerParams(needs_layout_passes=False,
                                             use_tc_tiling_on_sc=False),
    )
    out_flat = run(theta.reshape(-1), img_flat)
    return out_flat.reshape(B, H, W, C)


# speedup vs baseline: 1.0714x; 1.0714x over previous
import functools

import jax
import jax.numpy as jnp
from jax import lax
from jax.experimental import pallas as pl
from jax.experimental.pallas import tpu as pltpu
from jax.experimental.pallas import tpu_sc as plsc

_L = 16
_T = 128
_NB = 2
_SENT = -1


def _bf16_round(v):
    b = jax.lax.bitcast_convert_type(v, jnp.uint32)
    b = (b + jnp.uint32(0x7FFF) + ((b >> jnp.uint32(16)) & jnp.uint32(1)))
    b = b & jnp.uint32(0xFFFF0000)
    return jax.lax.bitcast_convert_type(b, jnp.float32)


def _sampler_body(H, W, C, rows_per_w, n_tiles,
                  theta_hbm, img_hbm, out_hbm,
                  theta_v, out_v, *bufs):
    HW = H * W
    per_set = 12
    sets = []
    for s in range(_NB):
        grp = bufs[s * per_set:(s + 1) * per_set]
        sets.append((grp[0:4], grp[4:8], grp[8:12], bufs[_NB * per_set + s]))

    wid = lax.axis_index("s") * 2 + lax.axis_index("c")
    row0 = wid * rows_per_w
    b = row0 // HW
    base = b * HW
    p0 = row0 - base

    pltpu.sync_copy(theta_hbm, theta_v)
    tbase = jnp.full((16,), b * 6, jnp.int32)

    def _tsplat(k):
        return _bf16_round(plsc.load_gather(theta_v, [tbase + k]))

    t00 = _tsplat(0)
    t01 = _tsplat(1)
    t02 = _tsplat(2)
    t10 = _tsplat(3)
    t11 = _tsplat(4)
    t12 = _tsplat(5)

    def compute_iw(g, s):
        idx, wts, _, _ = sets[s]
        ia_i, ib_i, ic_i, id_i = idx
        wa_v, wb_v, wc_v, wd_v = wts
        pstart = p0 + g * _T
        for u in range(_T // _L):
            p = (pstart + u * _L) + lax.iota(jnp.int32, 16)
            i = p // W
            j = p - i * W
            xn = _bf16_round(j.astype(jnp.float32) * jnp.float32(2.0 / (W - 1)) - 1.0)
            yn = _bf16_round(i.astype(jnp.float32) * jnp.float32(2.0 / (H - 1)) - 1.0)
            xs = t00 * xn + t01 * yn + t02
            ys = t10 * xn + t11 * yn + t12
            x = 0.5 * (xs + 1.0) * jnp.float32(W - 1)
            y = 0.5 * (ys + 1.0) * jnp.float32(H - 1)
            xt = x.astype(jnp.int32)
            yt = y.astype(jnp.int32)
            x0i = jnp.where(x < xt.astype(jnp.float32), xt - 1, xt)
            y0i = jnp.where(y < yt.astype(jnp.float32), yt - 1, yt)
            x0f = x0i.astype(jnp.float32)
            y0f = y0i.astype(jnp.float32)
            x0c = jnp.clip(x0i, 0, W - 1)
            x1c = jnp.clip(x0i + 1, 0, W - 1)
            y0c = jnp.clip(y0i, 0, H - 1)
            y1c = jnp.clip(y0i + 1, 0, H - 1)
            wx1 = (x0f + 1.0) - x
            wx0 = x - x0f
            wy1 = (y0f + 1.0) - y
            wy0 = y - y0f
            wa = wx1 * wy1
            wb = wx1 * wy0
            wc = wx0 * wy1
            wd = wx0 * wy0
            xcl = (x0i < 0) | (x0i >= W - 1)
            ycl = (y0i < 0) | (y0i >= H - 1)
            zero = jnp.zeros((16,), jnp.float32)
            wb_x = wb + jnp.where(xcl, wd, zero)
            wa_f = (wa + jnp.where(xcl, wc, zero)
                    + jnp.where(ycl, wb_x, zero))
            wb_f = jnp.where(ycl, zero, wb_x)
            wc_f = jnp.where(xcl, zero, wc + jnp.where(ycl, wd, zero))
            wd_f = jnp.where(xcl | ycl, zero, wd)
            sent = jnp.full((16,), _SENT, jnp.int32)
            sl = pl.ds(u * _L, _L)
            ia_i[sl] = base + y0c * W + x0c
            ib_i[sl] = jnp.where(ycl, sent, base + y1c * W + x0c)
            ic_i[sl] = jnp.where(xcl, sent, base + y0c * W + x1c)
            id_i[sl] = jnp.where(xcl | ycl, sent, base + y1c * W + x1c)
            wa_v[sl] = wa_f
            wb_v[sl] = wb_f
            wc_v[sl] = wc_f
            wd_v[sl] = wd_f

    def fire(s):
        idx, _, gat, sm = sets[s]
        for k in range(4):
            src = img_hbm.at[plsc.Indices(idx[k], ignored_value=_SENT)]
            pltpu.async_copy(src, gat[k], sm)

    def drain(s):
        idx, _, gat, sm = sets[s]
        for k in range(4):
            src = img_hbm.at[plsc.Indices(idx[k], ignored_value=_SENT)]
            pltpu.make_async_copy(src, gat[k], sm).wait()

    def combine_and_store(g, s):
        _, wts, gat, _ = sets[s]
        wa_v, wb_v, wc_v, wd_v = wts
        ga_v, gb_v, gc_v, gd_v = gat

        def row_body(t, c2):
            tt = jnp.full((16,), t, jnp.int32)
            wa = plsc.load_gather(wa_v, [tt])
            wb = plsc.load_gather(wb_v, [tt])
            wc = plsc.load_gather(wc_v, [tt])
            wd = plsc.load_gather(wd_v, [tt])
            for c0 in range(C // _L):
                cs = pl.ds(c0 * _L, _L)
                out_v[t, cs] = (wa * ga_v[t, cs] + wb * gb_v[t, cs]
                                + wc * gc_v[t, cs] + wd * gd_v[t, cs])
            return c2

        lax.fori_loop(0, _T, row_body, 0, unroll=2)
        pltpu.sync_copy(out_v, out_hbm.at[pl.ds(row0 + g * _T, _T)])

    def zero_body(t, c2):
        for s in range(_NB):
            for k in range(4):
                for c0 in range(C // _L):
                    sets[s][2][k][t, pl.ds(c0 * _L, _L)] = (
                        jnp.zeros((16,), jnp.float32))
        return c2

    lax.fori_loop(0, _T, zero_body, 0)

    compute_iw(0, 0)
    fire(0)

    def pair_body(gg, carry):
        g = gg * 2
        compute_iw(g + 1, 1)
        fire(1)
        drain(0)
        combine_and_store(g, 0)

        @pl.when(g + 2 < n_tiles)
        def _():
            compute_iw(g + 2, 0)
            fire(0)

        drain(1)
        combine_and_store(g + 1, 1)
        return carry

    lax.fori_loop(0, n_tiles // 2, pair_body, 0)


def kernel(theta, image):
    B, H, W, C = image.shape
    info = plsc.get_sparse_core_info()
    nw = info.num_cores * info.num_subcores
    total = B * H * W
    assert total % (nw * _T) == 0 and C % _L == 0
    rows_per_w = total // nw
    n_tiles = rows_per_w // _T
    assert n_tiles % _NB == 0

    img_flat = image.reshape(total, C)
    mesh = plsc.VectorSubcoreMesh(core_axis_name="c", subcore_axis_name="s")

    def bufset():
        return ([pltpu.VMEM((_T,), jnp.int32) for _ in range(4)]
                + [pltpu.VMEM((_T,), jnp.float32) for _ in range(4)]
                + [pltpu.VMEM((_T, C), jnp.float32) for _ in range(4)])

    scratch = [pltpu.VMEM((B * 6,), jnp.float32),
               pltpu.VMEM((_T, C), jnp.float32)]
    for _ in range(_NB):
        scratch += bufset()
    scratch += [pltpu.SemaphoreType.DMA] * _NB

    run = pl.kernel(
        functools.partial(_sampler_body, H, W, C, rows_per_w, n_tiles),
        out_type=jax.ShapeDtypeStruct((total, C), jnp.float32),
        mesh=mesh,
        scratch_types=scratch,
        compiler_params=pltpu.CompilerParams(needs_layout_passes=False,
                                             use_tc_tiling_on_sc=False),
    )
    out_flat = run(theta.reshape(-1), img_flat)
    return out_flat.reshape(B, H, W, C)


# speedup vs baseline: 1.0733x; 1.0018x over previous
import functools

import jax
import jax.numpy as jnp
from jax import lax
from jax.experimental import pallas as pl
from jax.experimental.pallas import tpu as pltpu
from jax.experimental.pallas import tpu_sc as plsc

_L = 16
_T = 128
_NB = 2
_SENT = -1


def _bf16_round(v):
    b = jax.lax.bitcast_convert_type(v, jnp.uint32)
    b = (b + jnp.uint32(0x7FFF) + ((b >> jnp.uint32(16)) & jnp.uint32(1)))
    b = b & jnp.uint32(0xFFFF0000)
    return jax.lax.bitcast_convert_type(b, jnp.float32)


def _sampler_body(H, W, C, rows_per_w, n_tiles,
                  theta_hbm, img_hbm, out_hbm,
                  theta_v, out_v, *bufs):
    HW = H * W
    per_set = 12
    sets = []
    for s in range(_NB):
        grp = bufs[s * per_set:(s + 1) * per_set]
        sets.append((grp[0:4], grp[4:8], grp[8:12], bufs[_NB * per_set + s]))

    wid = lax.axis_index("s") * 2 + lax.axis_index("c")
    row0 = wid * rows_per_w
    b = row0 // HW
    base = b * HW
    p0 = row0 - base

    pltpu.sync_copy(theta_hbm, theta_v)
    tbase = jnp.full((16,), b * 6, jnp.int32)

    def _tsplat(k):
        return _bf16_round(plsc.load_gather(theta_v, [tbase + k]))

    t00 = _tsplat(0)
    t01 = _tsplat(1)
    t02 = _tsplat(2)
    t10 = _tsplat(3)
    t11 = _tsplat(4)
    t12 = _tsplat(5)

    def compute_iw(g, s):
        idx, wts, _, _ = sets[s]
        ia_i, ib_i, ic_i, id_i = idx
        wa_v, wb_v, wc_v, wd_v = wts
        pstart = p0 + g * _T
        for u in range(_T // _L):
            p = (pstart + u * _L) + lax.iota(jnp.int32, 16)
            i = p // W
            j = p - i * W
            xn = _bf16_round(j.astype(jnp.float32) * jnp.float32(2.0 / (W - 1)) - 1.0)
            yn = _bf16_round(i.astype(jnp.float32) * jnp.float32(2.0 / (H - 1)) - 1.0)
            xs = t00 * xn + t01 * yn + t02
            ys = t10 * xn + t11 * yn + t12
            x = 0.5 * (xs + 1.0) * jnp.float32(W - 1)
            y = 0.5 * (ys + 1.0) * jnp.float32(H - 1)
            xt = x.astype(jnp.int32)
            yt = y.astype(jnp.int32)
            x0i = jnp.where(x < xt.astype(jnp.float32), xt - 1, xt)
            y0i = jnp.where(y < yt.astype(jnp.float32), yt - 1, yt)
            x0f = x0i.astype(jnp.float32)
            y0f = y0i.astype(jnp.float32)
            x0c = jnp.clip(x0i, 0, W - 1)
            x1c = jnp.clip(x0i + 1, 0, W - 1)
            y0c = jnp.clip(y0i, 0, H - 1)
            y1c = jnp.clip(y0i + 1, 0, H - 1)
            wx1 = (x0f + 1.0) - x
            wx0 = x - x0f
            wy1 = (y0f + 1.0) - y
            wy0 = y - y0f
            wa = wx1 * wy1
            wb = wx1 * wy0
            wc = wx0 * wy1
            wd = wx0 * wy0
            xcl = (x0i < 0) | (x0i >= W - 1)
            ycl = (y0i < 0) | (y0i >= H - 1)
            zero = jnp.zeros((16,), jnp.float32)
            wb_x = wb + jnp.where(xcl, wd, zero)
            wa_f = (wa + jnp.where(xcl, wc, zero)
                    + jnp.where(ycl, wb_x, zero))
            wb_f = jnp.where(ycl, zero, wb_x)
            wc_f = jnp.where(xcl, zero, wc + jnp.where(ycl, wd, zero))
            wd_f = jnp.where(xcl | ycl, zero, wd)
            sent = jnp.full((16,), _SENT, jnp.int32)
            sl = pl.ds(u * _L, _L)
            ia_i[sl] = y0c * W + x0c
            ib_i[sl] = jnp.where(ycl, sent, y1c * W + x0c)
            ic_i[sl] = jnp.where(xcl, sent, y0c * W + x1c)
            id_i[sl] = jnp.where(xcl | ycl, sent, y1c * W + x1c)
            wa_v[sl] = wa_f
            wb_v[sl] = wb_f
            wc_v[sl] = wc_f
            wd_v[sl] = wd_f

    img_b = img_hbm.at[b]

    def fire(s):
        idx, _, gat, sm = sets[s]
        for k in range(4):
            src = img_b.at[plsc.Indices(idx[k], ignored_value=_SENT)]
            pltpu.async_copy(src, gat[k], sm)

    def drain(s):
        idx, _, gat, sm = sets[s]
        for k in range(4):
            src = img_b.at[plsc.Indices(idx[k], ignored_value=_SENT)]
            pltpu.make_async_copy(src, gat[k], sm).wait()

    def combine_and_store(g, s):
        _, wts, gat, _ = sets[s]
        wa_v, wb_v, wc_v, wd_v = wts
        ga_v, gb_v, gc_v, gd_v = gat

        def row_body(t, c2):
            tt = jnp.full((16,), t, jnp.int32)
            wa = plsc.load_gather(wa_v, [tt])
            wb = plsc.load_gather(wb_v, [tt])
            wc = plsc.load_gather(wc_v, [tt])
            wd = plsc.load_gather(wd_v, [tt])
            for c0 in range(C // _L):
                cs = pl.ds(c0 * _L, _L)
                out_v[t, cs] = (wa * ga_v[t, cs] + wb * gb_v[t, cs]
                                + wc * gc_v[t, cs] + wd * gd_v[t, cs])
            return c2

        lax.fori_loop(0, _T, row_body, 0, unroll=2)
        pltpu.sync_copy(out_v, out_hbm.at[b, pl.ds(p0 + g * _T, _T)])

    def zero_body(t, c2):
        for s in range(_NB):
            for k in range(4):
                for c0 in range(C // _L):
                    sets[s][2][k][t, pl.ds(c0 * _L, _L)] = (
                        jnp.zeros((16,), jnp.float32))
        return c2

    lax.fori_loop(0, _T, zero_body, 0)

    compute_iw(0, 0)
    fire(0)

    def pair_body(gg, carry):
        g = gg * 2
        compute_iw(g + 1, 1)
        fire(1)
        drain(0)
        combine_and_store(g, 0)

        @pl.when(g + 2 < n_tiles)
        def _():
            compute_iw(g + 2, 0)
            fire(0)

        drain(1)
        combine_and_store(g + 1, 1)
        return carry

    lax.fori_loop(0, n_tiles // 2, pair_body, 0)


def kernel(theta, image):
    B, H, W, C = image.shape
    info = plsc.get_sparse_core_info()
    nw = info.num_cores * info.num_subcores
    total = B * H * W
    assert total % (nw * _T) == 0 and C % _L == 0
    rows_per_w = total // nw
    n_tiles = rows_per_w // _T
    assert n_tiles % _NB == 0

    img_flat = image.reshape(B, H * W, C)
    mesh = plsc.VectorSubcoreMesh(core_axis_name="c", subcore_axis_name="s")

    def bufset():
        return ([pltpu.VMEM((_T,), jnp.int32) for _ in range(4)]
                + [pltpu.VMEM((_T,), jnp.float32) for _ in range(4)]
                + [pltpu.VMEM((_T, C), jnp.float32) for _ in range(4)])

    scratch = [pltpu.VMEM((B * 6,), jnp.float32),
               pltpu.VMEM((_T, C), jnp.float32)]
    for _ in range(_NB):
        scratch += bufset()
    scratch += [pltpu.SemaphoreType.DMA] * _NB

    run = pl.kernel(
        functools.partial(_sampler_body, H, W, C, rows_per_w, n_tiles),
        out_type=jax.ShapeDtypeStruct((B, H * W, C), jnp.float32),
        mesh=mesh,
        scratch_types=scratch,
        compiler_params=pltpu.CompilerParams(needs_layout_passes=False,
                                             use_tc_tiling_on_sc=False),
    )
    out_flat = run(theta.reshape(-1), img_flat)
    return out_flat.reshape(B, H, W, C)


# speedup vs baseline: 1.0743x; 1.0010x over previous
import functools

import jax
import jax.numpy as jnp
from jax import lax
from jax.experimental import pallas as pl
from jax.experimental.pallas import tpu as pltpu
from jax.experimental.pallas import tpu_sc as plsc

_L = 16
_T = 128
_NB = 2
_SENT = -1


def _bf16_round(v):
    b = jax.lax.bitcast_convert_type(v, jnp.uint32)
    b = (b + jnp.uint32(0x7FFF) + ((b >> jnp.uint32(16)) & jnp.uint32(1)))
    b = b & jnp.uint32(0xFFFF0000)
    return jax.lax.bitcast_convert_type(b, jnp.float32)


def _sampler_body(H, W, C, rows_per_w, n_tiles,
                  theta_hbm, img_hbm, out_hbm,
                  theta_v, out_v, *bufs):
    HW = H * W
    per_set = 12
    sets = []
    for s in range(_NB):
        grp = bufs[s * per_set:(s + 1) * per_set]
        sets.append((grp[0:4], grp[4:8], grp[8:12], bufs[_NB * per_set + s]))

    wid = lax.axis_index("s") * 2 + lax.axis_index("c")
    row0 = wid * rows_per_w
    b = row0 // HW
    base = b * HW
    p0 = row0 - base

    pltpu.sync_copy(theta_hbm, theta_v)
    tbase = jnp.full((16,), b * 6, jnp.int32)

    def _tsplat(k):
        return _bf16_round(plsc.load_gather(theta_v, [tbase + k]))

    t00 = _tsplat(0)
    t01 = _tsplat(1)
    t02 = _tsplat(2)
    t10 = _tsplat(3)
    t11 = _tsplat(4)
    t12 = _tsplat(5)

    def compute_iw(g, s):
        idx, wts, _, _ = sets[s]
        ia_i, ib_i, ic_i, id_i = idx
        wa_v, wb_v, wc_v, wd_v = wts
        pstart = p0 + g * _T
        for u in range(_T // _L):
            p = (pstart + u * _L) + lax.iota(jnp.int32, 16)
            i = p // W
            j = p - i * W
            xn = _bf16_round(j.astype(jnp.float32) * jnp.float32(2.0 / (W - 1)) - 1.0)
            yn = _bf16_round(i.astype(jnp.float32) * jnp.float32(2.0 / (H - 1)) - 1.0)
            xs = t00 * xn + t01 * yn + t02
            ys = t10 * xn + t11 * yn + t12
            x = 0.5 * (xs + 1.0) * jnp.float32(W - 1)
            y = 0.5 * (ys + 1.0) * jnp.float32(H - 1)
            xt = x.astype(jnp.int32)
            yt = y.astype(jnp.int32)
            x0i = jnp.where(x < xt.astype(jnp.float32), xt - 1, xt)
            y0i = jnp.where(y < yt.astype(jnp.float32), yt - 1, yt)
            x0f = x0i.astype(jnp.float32)
            y0f = y0i.astype(jnp.float32)
            x0c = jnp.clip(x0i, 0, W - 1)
            x1c = jnp.clip(x0i + 1, 0, W - 1)
            y0c = jnp.clip(y0i, 0, H - 1)
            y1c = jnp.clip(y0i + 1, 0, H - 1)
            wx1 = (x0f + 1.0) - x
            wx0 = x - x0f
            wy1 = (y0f + 1.0) - y
            wy0 = y - y0f
            wa = wx1 * wy1
            wb = wx1 * wy0
            wc = wx0 * wy1
            wd = wx0 * wy0
            xcl = (x0i < 0) | (x0i >= W - 1)
            ycl = (y0i < 0) | (y0i >= H - 1)
            zero = jnp.zeros((16,), jnp.float32)
            wb_x = wb + jnp.where(xcl, wd, zero)
            wa_f = (wa + jnp.where(xcl, wc, zero)
                    + jnp.where(ycl, wb_x, zero))
            wb_f = jnp.where(ycl, zero, wb_x)
            wc_f = jnp.where(xcl, zero, wc + jnp.where(ycl, wd, zero))
            wd_f = jnp.where(xcl | ycl, zero, wd)
            sent = jnp.full((16,), _SENT, jnp.int32)
            sl = pl.ds(u * _L, _L)
            ia_i[sl] = y0c * W + x0c
            ib_i[sl] = jnp.where(ycl, sent, y1c * W + x0c)
            ic_i[sl] = jnp.where(xcl, sent, y0c * W + x1c)
            id_i[sl] = jnp.where(xcl | ycl, sent, y1c * W + x1c)
            wa_v[sl] = wa_f
            wb_v[sl] = wb_f
            wc_v[sl] = wc_f
            wd_v[sl] = wd_f

    img_b = img_hbm.at[b]

    def fire(s):
        idx, _, gat, sm = sets[s]
        for k in range(4):
            src = img_b.at[plsc.Indices(idx[k], ignored_value=_SENT)]
            pltpu.async_copy(src, gat[k], sm)

    def drain(s):
        idx, _, gat, sm = sets[s]
        for k in range(4):
            src = img_b.at[plsc.Indices(idx[k], ignored_value=_SENT)]
            pltpu.make_async_copy(src, gat[k], sm).wait()

    def combine_and_store(g, s):
        _, wts, gat, _ = sets[s]
        wa_v, wb_v, wc_v, wd_v = wts
        ga_v, gb_v, gc_v, gd_v = gat

        def row_body(t, c2):
            tt = jnp.full((16,), t, jnp.int32)
            wa = plsc.load_gather(wa_v, [tt])
            wb = plsc.load_gather(wb_v, [tt])
            wc = plsc.load_gather(wc_v, [tt])
            wd = plsc.load_gather(wd_v, [tt])
            for c0 in range(C // _L):
                cs = pl.ds(c0 * _L, _L)
                out_v[t, cs] = (wa * ga_v[t, cs] + wb * gb_v[t, cs]
                                + wc * gc_v[t, cs] + wd * gd_v[t, cs])
            return c2

        lax.fori_loop(0, _T, row_body, 0, unroll=2)
        pltpu.sync_copy(out_v, out_hbm.at[b, pl.ds(p0 + g * _T, _T)])

    def zero_body(t, c2):
        for s in range(_NB):
            for k in range(4):
                for c0 in range(C // _L):
                    sets[s][2][k][t, pl.ds(c0 * _L, _L)] = (
                        jnp.zeros((16,), jnp.float32))
        return c2

    lax.fori_loop(0, _T, zero_body, 0)

    compute_iw(0, 0)
    fire(0)

    def pair_body(gg, carry):
        g = gg * 2
        compute_iw(g + 1, 1)
        fire(1)
        drain(0)
        combine_and_store(g, 0)

        @pl.when(g + 2 < n_tiles)
        def _():
            compute_iw(g + 2, 0)
            fire(0)

        drain(1)
        combine_and_store(g + 1, 1)
        return carry

    lax.fori_loop(0, n_tiles // 2, pair_body, 0)


def kernel(theta, image):
    B, H, W, C = image.shape
    info = plsc.get_sparse_core_info()
    nw = info.num_cores * info.num_subcores
    total = B * H * W
    assert total % (nw * _T) == 0 and C % _L == 0
    assert (H * W) % (total // nw) == 0
    rows_per_w = total // nw
    n_tiles = rows_per_w // _T
    assert n_tiles % _NB == 0

    img_flat = image.reshape(B, H * W, C)
    mesh = plsc.VectorSubcoreMesh(core_axis_name="c", subcore_axis_name="s")

    def bufset():
        return ([pltpu.VMEM((_T,), jnp.int32) for _ in range(4)]
                + [pltpu.VMEM((_T,), jnp.float32) for _ in range(4)]
                + [pltpu.VMEM((_T, C), jnp.float32) for _ in range(4)])

    scratch = [pltpu.VMEM((B * 6,), jnp.float32),
               pltpu.VMEM((_T, C), jnp.float32)]
    for _ in range(_NB):
        scratch += bufset()
    scratch += [pltpu.SemaphoreType.DMA] * _NB

    run = pl.kernel(
        functools.partial(_sampler_body, H, W, C, rows_per_w, n_tiles),
        out_type=jax.ShapeDtypeStruct((B, H * W, C), jnp.float32),
        mesh=mesh,
        scratch_types=scratch,
        compiler_params=pltpu.CompilerParams(needs_layout_passes=False,
                                             use_tc_tiling_on_sc=False),
    )
    out_flat = run(theta.reshape(-1), img_flat)
    return out_flat.reshape(B, H, W, C)
